# baseline (speedup 1.0000x reference)
import jax
import jax.numpy as jnp
from jax.experimental import pallas as pl
from jax.experimental.pallas import tpu as pltpu

_K = 3
_BI = 512
_WPAD = 40


def _tagconv_body(a_ref, x0t_ref, winit_ref, wmid_ref, out_ref,
                  a_scr, xbuf, acc1, ystash):
    n = a_ref.shape[1]
    f = winit_ref.shape[0]
    nblk = n // _BI
    k = pl.program_id(0)
    i = pl.program_id(1)
    row = pl.ds(i * _BI, _BI)

    @pl.when(jnp.logical_and(k == 0, i == 0))
    def _init():
        z = jnp.dot(winit_ref[...], x0t_ref[...],
                    preferred_element_type=jnp.float32)
        ones = jnp.ones((1, n), jnp.float32)
        zeros = jnp.zeros((_WPAD - f - 1, n), jnp.float32)
        xbuf[0] = jnp.concatenate([z, ones, zeros], axis=0).astype(jnp.bfloat16)
        acc1[...] = jnp.zeros((_WPAD, n), jnp.float32)

    def _chunk(rd, r0, clen):
        rows = pl.ds(r0, clen)
        a = a_scr[rows, :]
        y = jax.lax.dot_general(xbuf[rd], a, (((1,), (1,)), ((), ())),
                                preferred_element_type=jnp.float32)
        y = y + xbuf[rd, :, rows].astype(jnp.float32)
        rs = y[f:f + 1, :]
        rs = jnp.where(rs == 0.0, 1.0, rs)
        return y / rs

    def _wadd(wsel, r0, clen):
        return jnp.dot(wmid_ref[wsel], x0t_ref[:, pl.ds(r0, clen)],
                       preferred_element_type=jnp.float32)

    def _acc_tile(rb, jb):
        jcols = pl.ds(jb * _BI, _BI)
        rrows = pl.ds(rb * _BI, _BI)
        part = jax.lax.dot_general(
            xbuf[1, :, jcols], a_scr[rrows, jcols], (((1,), (1,)), ((), ())),
            preferred_element_type=jnp.float32)
        acc1[:, rrows] += part

    @pl.when(k == 0)
    def _stage_hop0_and_band():
        a_scr[row, :] = a_ref[...].astype(jnp.bfloat16)
        y = _chunk(0, i * _BI, _BI)
        xbuf[1, :, row] = (y + _wadd(0, i * _BI, _BI)).astype(jnp.bfloat16)
        for d in range(2 * (nblk - 1) - 1):
            @pl.when(d < 2 * i - 1)
            def _band_tile():
                rb = jnp.where(d < i - 1, d, i - 1)
                jb = jnp.where(d < i - 1, i - 1, d - (i - 1))
                _acc_tile(rb, jb)

    @pl.when(jnp.logical_and(k == 1, i == 0))
    def _hop1_finish():
        for r in range(nblk - 1):
            _acc_tile(r, nblk - 1)
        for j in range(nblk):
            _acc_tile(nblk - 1, j)
        for c in range(nblk):
            rows = pl.ds(c * _BI, _BI)
            y = acc1[:, rows] + xbuf[1, :, rows].astype(jnp.float32)
            rs = y[f:f + 1, :]
            rs = jnp.where(rs == 0.0, 1.0, rs)
            y = y / rs
            xbuf[0, :, rows] = (y + _wadd(1, c * _BI, _BI)).astype(jnp.bfloat16)

    @pl.when(jnp.logical_and(k == 2, i == 0))
    def _hop2():
        for c in range(nblk):
            y = _chunk(0, c * _BI, _BI)
            ystash[:, pl.ds(c * _BI, _BI)] = y[:f, :]

    out_ref[...] = ystash[:, row]


def kernel(adjacency_matrices, weights_matrix, data, W):
    del weights_matrix
    n = adjacency_matrices.shape[-1]
    c, f = W.shape[0], W.shape[1]
    nblk = n // _BI
    pad = jnp.zeros((c, _WPAD - f), jnp.float32)
    wmid_t = jnp.stack([
        jnp.concatenate([W[:, :, 2], pad], axis=1).T,
        jnp.concatenate([W[:, :, 0] + W[:, :, 1], pad], axis=1).T,
    ])
    winit_t = W[:, :, 3].T
    x0_t = data.T

    out_t = pl.pallas_call(
        _tagconv_body,
        grid=(_K, nblk),
        in_specs=[
            pl.BlockSpec((_BI, n),
                         lambda k, i: (jnp.where(k == 0, i, nblk - 1), 0)),
            pl.BlockSpec((c, n), lambda k, i: (0, 0)),
            pl.BlockSpec((f, c), lambda k, i: (0, 0)),
            pl.BlockSpec((2, _WPAD, c), lambda k, i: (0, 0, 0)),
        ],
        out_specs=pl.BlockSpec((f, _BI), lambda k, i: (0, i)),
        out_shape=jax.ShapeDtypeStruct((f, n), jnp.float32),
        scratch_shapes=[
            pltpu.VMEM((n, n), jnp.bfloat16),
            pltpu.VMEM((2, _WPAD, n), jnp.bfloat16),
            pltpu.VMEM((_WPAD, n), jnp.float32),
            pltpu.VMEM((f, n), jnp.float32),
        ],
    )(adjacency_matrices, x0_t, winit_t, wmid_t)
    return out_t.T


# speedup vs baseline: 1.0301x; 1.0301x over previous
import jax
import jax.numpy as jnp
from jax.experimental import pallas as pl
from jax.experimental.pallas import tpu as pltpu

_K = 3
_BI = 512
_WPAD = 40


def _tagconv_body(a_ref, x0t_ref, winit_ref, wmid_ref, out_ref,
                  a_scr, xbuf, ystash):
    n = a_ref.shape[1]
    f = winit_ref.shape[0]
    nblk = n // _BI
    k = pl.program_id(0)
    i = pl.program_id(1)
    row = pl.ds(i * _BI, _BI)

    @pl.when(jnp.logical_and(k == 0, i == 0))
    def _init():
        z = jnp.dot(winit_ref[...], x0t_ref[...],
                    preferred_element_type=jnp.float32)
        ones = jnp.ones((1, n), jnp.float32)
        zeros = jnp.zeros((_WPAD - f - 1, n), jnp.float32)
        xbuf[0] = jnp.concatenate([z, ones, zeros], axis=0).astype(jnp.bfloat16)

    def _chunk(rd, r0, clen):
        rows = pl.ds(r0, clen)
        a = a_scr[rows, :]
        y = jax.lax.dot_general(xbuf[rd], a, (((1,), (1,)), ((), ())),
                                preferred_element_type=jnp.float32)
        y = y + xbuf[rd, :, rows].astype(jnp.float32)
        rs = y[f:f + 1, :]
        rs = jnp.where(rs == 0.0, 1.0, rs)
        return y / rs

    def _wadd(wsel, r0, clen):
        return jnp.dot(wmid_ref[wsel], x0t_ref[:, pl.ds(r0, clen)],
                       preferred_element_type=jnp.float32)

    @pl.when(k == 0)
    def _stage_and_hop0():
        a_scr[row, :] = a_ref[...].astype(jnp.bfloat16)
        y = _chunk(0, i * _BI, _BI)
        xbuf[1, :, row] = (y + _wadd(0, i * _BI, _BI)).astype(jnp.bfloat16)

    @pl.when(jnp.logical_and(k == 1, i == 0))
    def _hop1():
        for c in range(nblk):
            y = _chunk(1, c * _BI, _BI)
            xbuf[0, :, pl.ds(c * _BI, _BI)] = \
                (y + _wadd(1, c * _BI, _BI)).astype(jnp.bfloat16)

    @pl.when(jnp.logical_and(k == 2, i == 0))
    def _hop2():
        for c in range(nblk):
            y = _chunk(0, c * _BI, _BI)
            ystash[:, pl.ds(c * _BI, _BI)] = y[:f, :]

    out_ref[...] = ystash[:, row]


def kernel(adjacency_matrices, weights_matrix, data, W):
    del weights_matrix
    n = adjacency_matrices.shape[-1]
    c, f = W.shape[0], W.shape[1]
    nblk = n // _BI
    pad = jnp.zeros((c, _WPAD - f), jnp.float32)
    wmid_t = jnp.stack([
        jnp.concatenate([W[:, :, 2], pad], axis=1).T,
        jnp.concatenate([W[:, :, 0] + W[:, :, 1], pad], axis=1).T,
    ])
    winit_t = W[:, :, 3].T
    x0_t = data.T

    out_t = pl.pallas_call(
        _tagconv_body,
        grid=(_K, nblk),
        in_specs=[
            pl.BlockSpec((_BI, n),
                         lambda k, i: (jnp.where(k == 0, i, nblk - 1), 0)),
            pl.BlockSpec((c, n), lambda k, i: (0, 0)),
            pl.BlockSpec((f, c), lambda k, i: (0, 0)),
            pl.BlockSpec((2, _WPAD, c), lambda k, i: (0, 0, 0)),
        ],
        out_specs=pl.BlockSpec((f, _BI), lambda k, i: (0, i)),
        out_shape=jax.ShapeDtypeStruct((f, n), jnp.float32),
        scratch_shapes=[
            pltpu.VMEM((n, n), jnp.bfloat16),
            pltpu.VMEM((2, _WPAD, n), jnp.bfloat16),
            pltpu.VMEM((f, n), jnp.float32),
        ],
    )(adjacency_matrices, x0_t, winit_t, wmid_t)
    return out_t.T
